# initial kernel scaffold (unmeasured)
import jax
import jax.numpy as jnp
from jax import lax
from jax.experimental import pallas as pl
from jax.experimental.pallas import tpu as pltpu

N_DEV = 16


def kernel(x, Wq, Wo, K_ext, V_ext):
    B, Sq, D = x.shape
    Dq = Wq.shape[1]
    _, Skv, Hq, Dh = K_ext.shape
    n_hops = N_DEV - 1

    def body(x_ref, wq_ref, wo_ref, k_ref, v_ref, out_ref,
             q_s, o_own, l_own, o_acc, l_acc, o_comm, l_comm,
             o_send_sems, o_recv_sems, l_send_sems, l_recv_sems):
        my_pos = lax.axis_index("i")
        left = lax.rem(my_pos - 1 + N_DEV, N_DEV)
        right = lax.rem(my_pos + 1, N_DEV)

        barrier_sem = pltpu.get_barrier_semaphore()
        for nbr in (left, right):
            pl.semaphore_signal(
                barrier_sem, inc=1,
                device_id=(nbr,), device_id_type=pl.DeviceIdType.MESH,
            )
        pl.semaphore_wait(barrier_sem, 2)

        for b in range(B):
            q_s[b] = jnp.dot(x_ref[b], wq_ref[...],
                             preferred_element_type=jnp.float32)

        ones = jnp.ones((Skv, 1), dtype=jnp.float32)
        for b in range(B):
            for h in range(Hq):
                qh = q_s[b, :, h * Dh:(h + 1) * Dh]
                kh = k_ref[b, :, h, :]
                vh = v_ref[b, :, h, :]
                s = lax.dot_general(
                    qh, kh, (((1,), (1,)), ((), ())),
                    preferred_element_type=jnp.float32,
                ) * 0.125
                p = jnp.exp(s)
                o_own[b, :, h * Dh:(h + 1) * Dh] = jnp.dot(
                    p, vh, preferred_element_type=jnp.float32)
                l_own[b, h] = jnp.dot(
                    p, ones, preferred_element_type=jnp.float32)

        o_acc[...] = o_own[...]
        l_acc[...] = l_own[...]

        for h in range(n_hops):
            o_src = o_own if h == 0 else o_comm.at[h - 1]
            l_src = l_own if h == 0 else l_comm.at[h - 1]
            o_rdma = pltpu.make_async_remote_copy(
                src_ref=o_src,
                dst_ref=o_comm.at[h],
                send_sem=o_send_sems.at[h],
                recv_sem=o_recv_sems.at[h],
                device_id=(right,),
                device_id_type=pl.DeviceIdType.MESH,
            )
            l_rdma = pltpu.make_async_remote_copy(
                src_ref=l_src,
                dst_ref=l_comm.at[h],
                send_sem=l_send_sems.at[h],
                recv_sem=l_recv_sems.at[h],
                device_id=(right,),
                device_id_type=pl.DeviceIdType.MESH,
            )
            o_rdma.start()
            l_rdma.start()
            o_rdma.wait()
            l_rdma.wait()
            o_acc[...] += o_comm[h]
            l_acc[...] += l_comm[h]

        for b in range(B):
            for h in range(Hq):
                q_s[b, :, h * Dh:(h + 1) * Dh] = (
                    o_acc[b, :, h * Dh:(h + 1) * Dh] / l_acc[b, h]
                )
        for b in range(B):
            out_ref[b] = jnp.dot(q_s[b], wo_ref[...],
                                 preferred_element_type=jnp.float32)

    return pl.pallas_call(
        body,
        out_shape=jax.ShapeDtypeStruct((B, Sq, D), jnp.float32),
        in_specs=[pl.BlockSpec(memory_space=pltpu.VMEM)] * 5,
        out_specs=pl.BlockSpec(memory_space=pltpu.VMEM),
        scratch_shapes=[
            pltpu.VMEM((B, Sq, Dq), jnp.float32),
            pltpu.VMEM((B, Sq, Dq), jnp.float32),
            pltpu.VMEM((B, Hq, Sq, 1), jnp.float32),
            pltpu.VMEM((B, Sq, Dq), jnp.float32),
            pltpu.VMEM((B, Hq, Sq, 1), jnp.float32),
            pltpu.VMEM((N_DEV - 1, B, Sq, Dq), jnp.float32),
            pltpu.VMEM((N_DEV - 1, B, Hq, Sq, 1), jnp.float32),
            pltpu.SemaphoreType.DMA((N_DEV - 1,)),
            pltpu.SemaphoreType.DMA((N_DEV - 1,)),
            pltpu.SemaphoreType.DMA((N_DEV - 1,)),
            pltpu.SemaphoreType.DMA((N_DEV - 1,)),
        ],
        compiler_params=pltpu.CompilerParams(collective_id=0),
    )(x, Wq, Wo, K_ext, V_ext)


# baseline (device time: 246664 ns/iter reference)
import jax
import jax.numpy as jnp
from jax import lax
from jax.experimental import pallas as pl
from jax.experimental.pallas import tpu as pltpu

N_DEV = 16


def kernel(x, Wq, Wo, K_ext, V_ext):
    B, Sq, D = x.shape
    Dq = Wq.shape[1]
    _, Skv, Hq, Dh = K_ext.shape
    n_hops = N_DEV - 1

    def body(x_ref, wq_ref, wo_ref, k_ref, v_ref, out_ref,
             q_s, o_own, l_own, o_acc, l_acc, o_comm, l_comm,
             o_send_sems, o_recv_sems, l_send_sems, l_recv_sems):
        my_pos = lax.axis_index("i")
        left = lax.rem(my_pos - 1 + N_DEV, N_DEV)
        right = lax.rem(my_pos + 1, N_DEV)

        barrier_sem = pltpu.get_barrier_semaphore()
        for nbr in (left, right):
            pl.semaphore_signal(
                barrier_sem, inc=1,
                device_id=(nbr,), device_id_type=pl.DeviceIdType.MESH,
            )
        pl.semaphore_wait(barrier_sem, 2)

        for b in range(B):
            q_s[b] = jnp.dot(x_ref[b], wq_ref[...],
                             preferred_element_type=jnp.float32)

        ones = jnp.ones((Skv, 1), dtype=jnp.float32)
        for b in range(B):
            for h in range(Hq):
                qh = q_s[b, :, h * Dh:(h + 1) * Dh]
                kh = k_ref[b, :, h, :]
                vh = v_ref[b, :, h, :]
                s = lax.dot_general(
                    qh, kh, (((1,), (1,)), ((), ())),
                    preferred_element_type=jnp.float32,
                ) * 0.125
                p = jnp.exp(s)
                o_own[b, :, h * Dh:(h + 1) * Dh] = jnp.dot(
                    p, vh, preferred_element_type=jnp.float32)
                c = b * Hq + h
                l_own[:, c:c + 1] = jnp.dot(
                    p, ones, preferred_element_type=jnp.float32)

        o_acc[...] = o_own[...]
        l_acc[...] = l_own[...]

        for h in range(n_hops):
            o_src = o_own if h == 0 else o_comm.at[h - 1]
            l_src = l_own if h == 0 else l_comm.at[h - 1]
            o_rdma = pltpu.make_async_remote_copy(
                src_ref=o_src,
                dst_ref=o_comm.at[h],
                send_sem=o_send_sems.at[h],
                recv_sem=o_recv_sems.at[h],
                device_id=(right,),
                device_id_type=pl.DeviceIdType.MESH,
            )
            l_rdma = pltpu.make_async_remote_copy(
                src_ref=l_src,
                dst_ref=l_comm.at[h],
                send_sem=l_send_sems.at[h],
                recv_sem=l_recv_sems.at[h],
                device_id=(right,),
                device_id_type=pl.DeviceIdType.MESH,
            )
            o_rdma.start()
            l_rdma.start()
            o_rdma.wait()
            l_rdma.wait()
            o_acc[...] += o_comm[h]
            l_acc[...] += l_comm[h]

        for b in range(B):
            for h in range(Hq):
                c = b * Hq + h
                q_s[b, :, h * Dh:(h + 1) * Dh] = (
                    o_acc[b, :, h * Dh:(h + 1) * Dh] / l_acc[:, c:c + 1]
                )
        for b in range(B):
            out_ref[b] = jnp.dot(q_s[b], wo_ref[...],
                                 preferred_element_type=jnp.float32)

    return pl.pallas_call(
        body,
        out_shape=jax.ShapeDtypeStruct((B, Sq, D), jnp.float32),
        in_specs=[pl.BlockSpec(memory_space=pltpu.VMEM)] * 5,
        out_specs=pl.BlockSpec(memory_space=pltpu.VMEM),
        scratch_shapes=[
            pltpu.VMEM((B, Sq, Dq), jnp.float32),
            pltpu.VMEM((B, Sq, Dq), jnp.float32),
            pltpu.VMEM((Sq, B * Hq), jnp.float32),
            pltpu.VMEM((B, Sq, Dq), jnp.float32),
            pltpu.VMEM((Sq, B * Hq), jnp.float32),
            pltpu.VMEM((N_DEV - 1, B, Sq, Dq), jnp.float32),
            pltpu.VMEM((N_DEV - 1, Sq, B * Hq), jnp.float32),
            pltpu.SemaphoreType.DMA((N_DEV - 1,)),
            pltpu.SemaphoreType.DMA((N_DEV - 1,)),
            pltpu.SemaphoreType.DMA((N_DEV - 1,)),
            pltpu.SemaphoreType.DMA((N_DEV - 1,)),
        ],
        compiler_params=pltpu.CompilerParams(
            collective_id=0, vmem_limit_bytes=110 * 1024 * 1024),
    )(x, Wq, Wo, K_ext, V_ext)


# device time: 109555 ns/iter; 2.2515x vs baseline; 2.2515x over previous
import jax
import jax.numpy as jnp
from jax import lax
from jax.experimental import pallas as pl
from jax.experimental.pallas import tpu as pltpu

N_DEV = 16


def kernel(x, Wq, Wo, K_ext, V_ext):
    B, Sq, D = x.shape
    Dq = Wq.shape[1]
    _, Skv, Hq, Dh = K_ext.shape
    n_hops = N_DEV - 1
    C = Sq // N_DEV

    def body(x_ref, wq_ref, wo_ref, k_ref, v_ref, out_ref,
             q_s, o_own, l_own, on_chunk, rs_o, rs_l,
             rso_send, rso_recv, rsl_send, rsl_recv, ag_send, ag_recv):
        my_pos = lax.axis_index("i")
        left = lax.rem(my_pos - 1 + N_DEV, N_DEV)
        right = lax.rem(my_pos + 1, N_DEV)

        barrier_sem = pltpu.get_barrier_semaphore()
        for nbr in (left, right):
            pl.semaphore_signal(
                barrier_sem, inc=1,
                device_id=(nbr,), device_id_type=pl.DeviceIdType.MESH,
            )
        pl.semaphore_wait(barrier_sem, 2)

        for b in range(B):
            q_s[b] = jnp.dot(x_ref[b], wq_ref[...],
                             preferred_element_type=jnp.float32)

        ones = jnp.ones((Skv, 1), dtype=jnp.float32)
        for b in range(B):
            for h in range(Hq):
                qh = q_s[b, :, h * Dh:(h + 1) * Dh]
                kh = k_ref[b, :, h, :]
                vh = v_ref[b, :, h, :]
                s = lax.dot_general(
                    qh, kh, (((1,), (1,)), ((), ())),
                    preferred_element_type=jnp.float32,
                ) * 0.125
                p = jnp.exp(s)
                o_own[b, :, h * Dh:(h + 1) * Dh] = jnp.dot(
                    p, vh, preferred_element_type=jnp.float32)
                c = b * Hq + h
                l_own[:, c:c + 1] = jnp.dot(
                    p, ones, preferred_element_type=jnp.float32)

        def chunk_start(cid):
            return lax.rem(cid + 2 * N_DEV, N_DEV) * C

        for t in range(n_hops):
            if t == 0:
                s0 = chunk_start(my_pos)
                o_src = o_own.at[:, pl.ds(s0, C), :]
                l_src = l_own.at[pl.ds(s0, C), :]
            else:
                o_src = rs_o.at[t - 1]
                l_src = rs_l.at[t - 1]
            o_rdma = pltpu.make_async_remote_copy(
                src_ref=o_src, dst_ref=rs_o.at[t],
                send_sem=rso_send.at[t], recv_sem=rso_recv.at[t],
                device_id=(right,), device_id_type=pl.DeviceIdType.MESH,
            )
            l_rdma = pltpu.make_async_remote_copy(
                src_ref=l_src, dst_ref=rs_l.at[t],
                send_sem=rsl_send.at[t], recv_sem=rsl_recv.at[t],
                device_id=(right,), device_id_type=pl.DeviceIdType.MESH,
            )
            o_rdma.start()
            l_rdma.start()
            o_rdma.wait()
            l_rdma.wait()
            sr = chunk_start(my_pos - 1 - t)
            rs_o[t] += o_own[:, pl.ds(sr, C), :]
            rs_l[t] += l_own[pl.ds(sr, C), :]

        for b in range(B):
            for h in range(Hq):
                c = b * Hq + h
                on_chunk[b, :, h * Dh:(h + 1) * Dh] = (
                    rs_o[n_hops - 1, b, :, h * Dh:(h + 1) * Dh]
                    / rs_l[n_hops - 1, :, c:c + 1]
                )
        own_start = chunk_start(my_pos + 1)
        for b in range(B):
            out_ref[b, pl.ds(own_start, C), :] = jnp.dot(
                on_chunk[b], wo_ref[...], preferred_element_type=jnp.float32)

        for t in range(n_hops):
            sa = chunk_start(my_pos + 1 - t)
            ag_rdma = pltpu.make_async_remote_copy(
                src_ref=out_ref.at[:, pl.ds(sa, C), :],
                dst_ref=out_ref.at[:, pl.ds(sa, C), :],
                send_sem=ag_send.at[t], recv_sem=ag_recv.at[t],
                device_id=(right,), device_id_type=pl.DeviceIdType.MESH,
            )
            ag_rdma.start()
            ag_rdma.wait()

    return pl.pallas_call(
        body,
        out_shape=jax.ShapeDtypeStruct((B, Sq, D), jnp.float32),
        in_specs=[pl.BlockSpec(memory_space=pltpu.VMEM)] * 5,
        out_specs=pl.BlockSpec(memory_space=pltpu.VMEM),
        scratch_shapes=[
            pltpu.VMEM((B, Sq, Dq), jnp.float32),
            pltpu.VMEM((B, Sq, Dq), jnp.float32),
            pltpu.VMEM((Sq, B * Hq), jnp.float32),
            pltpu.VMEM((B, C, Dq), jnp.float32),
            pltpu.VMEM((N_DEV - 1, B, C, Dq), jnp.float32),
            pltpu.VMEM((N_DEV - 1, C, B * Hq), jnp.float32),
            pltpu.SemaphoreType.DMA((N_DEV - 1,)),
            pltpu.SemaphoreType.DMA((N_DEV - 1,)),
            pltpu.SemaphoreType.DMA((N_DEV - 1,)),
            pltpu.SemaphoreType.DMA((N_DEV - 1,)),
            pltpu.SemaphoreType.DMA((N_DEV - 1,)),
            pltpu.SemaphoreType.DMA((N_DEV - 1,)),
        ],
        compiler_params=pltpu.CompilerParams(
            collective_id=0, vmem_limit_bytes=110 * 1024 * 1024),
    )(x, Wq, Wo, K_ext, V_ext)


# device time: 56990 ns/iter; 4.3282x vs baseline; 1.9224x over previous
import jax
import jax.numpy as jnp
from jax import lax
from jax.experimental import pallas as pl
from jax.experimental.pallas import tpu as pltpu

N_DEV = 16


def kernel(x, Wq, Wo, K_ext, V_ext):
    B, Sq, D = x.shape
    Dq = Wq.shape[1]
    _, Skv, Hq, Dh = K_ext.shape
    C = Sq // N_DEV

    def body(x_ref, wq_ref, wo_ref, k_ref, v_ref, out_ref,
             q_s, o_own, l_own, on_chunk, rs_o, rs_l,
             rso_send, rso_recv, rsl_send, rsl_recv, ag_send, ag_recv):
        my_pos = lax.axis_index("i")

        barrier_sem = pltpu.get_barrier_semaphore()
        for nbr in range(N_DEV):
            @pl.when(nbr != my_pos)
            def _():
                pl.semaphore_signal(
                    barrier_sem, inc=1,
                    device_id=(nbr,), device_id_type=pl.DeviceIdType.MESH,
                )
        pl.semaphore_wait(barrier_sem, N_DEV - 1)

        for b in range(B):
            q_s[b] = jnp.dot(x_ref[b], wq_ref[...],
                             preferred_element_type=jnp.float32)

        ones = jnp.ones((Skv, 1), dtype=jnp.float32)
        for b in range(B):
            for h in range(Hq):
                qh = q_s[b, :, h * Dh:(h + 1) * Dh]
                kh = k_ref[b, :, h, :]
                vh = v_ref[b, :, h, :]
                s = lax.dot_general(
                    qh, kh, (((1,), (1,)), ((), ())),
                    preferred_element_type=jnp.float32,
                ) * 0.125
                p = jnp.exp(s)
                o_own[b, :, h * Dh:(h + 1) * Dh] = jnp.dot(
                    p, vh, preferred_element_type=jnp.float32)
                c = b * Hq + h
                l_own[:, c:c + 1] = jnp.dot(
                    p, ones, preferred_element_type=jnp.float32)

        rs_rdmas = []
        for d in range(N_DEV):
            o_rdma = pltpu.make_async_remote_copy(
                src_ref=o_own.at[:, pl.ds(d * C, C), :],
                dst_ref=rs_o.at[my_pos],
                send_sem=rso_send.at[d], recv_sem=rso_recv.at[my_pos],
                device_id=(d,), device_id_type=pl.DeviceIdType.MESH,
            )
            l_rdma = pltpu.make_async_remote_copy(
                src_ref=l_own.at[pl.ds(d * C, C), :],
                dst_ref=rs_l.at[my_pos],
                send_sem=rsl_send.at[d], recv_sem=rsl_recv.at[my_pos],
                device_id=(d,), device_id_type=pl.DeviceIdType.MESH,
            )

            @pl.when(d != my_pos)
            def _():
                o_rdma.start()
                l_rdma.start()

            rs_rdmas.append((d, o_rdma, l_rdma))

        rs_o[my_pos] = o_own[:, pl.ds(my_pos * C, C), :]
        rs_l[my_pos] = l_own[pl.ds(my_pos * C, C), :]

        for s in range(N_DEV):
            o_rx = pltpu.make_async_remote_copy(
                src_ref=rs_o.at[s], dst_ref=rs_o.at[s],
                send_sem=rso_send.at[s], recv_sem=rso_recv.at[s],
                device_id=(s,), device_id_type=pl.DeviceIdType.MESH,
            )
            l_rx = pltpu.make_async_remote_copy(
                src_ref=rs_l.at[s], dst_ref=rs_l.at[s],
                send_sem=rsl_send.at[s], recv_sem=rsl_recv.at[s],
                device_id=(s,), device_id_type=pl.DeviceIdType.MESH,
            )

            @pl.when(s != my_pos)
            def _():
                o_rx.wait_recv()
                l_rx.wait_recv()

        on_chunk[...] = rs_o[0]
        for s in range(1, N_DEV):
            on_chunk[...] += rs_o[s]
        l_tot = rs_l[0]
        for s in range(1, N_DEV):
            l_tot = l_tot + rs_l[s]

        for b in range(B):
            for h in range(Hq):
                c = b * Hq + h
                on_chunk[b, :, h * Dh:(h + 1) * Dh] = (
                    on_chunk[b, :, h * Dh:(h + 1) * Dh] / l_tot[:, c:c + 1]
                )
        for b in range(B):
            out_ref[b, pl.ds(my_pos * C, C), :] = jnp.dot(
                on_chunk[b], wo_ref[...], preferred_element_type=jnp.float32)

        ag_rdmas = []
        for d in range(N_DEV):
            ag_rdma = pltpu.make_async_remote_copy(
                src_ref=out_ref.at[:, pl.ds(my_pos * C, C), :],
                dst_ref=out_ref.at[:, pl.ds(my_pos * C, C), :],
                send_sem=ag_send.at[d], recv_sem=ag_recv.at[my_pos],
                device_id=(d,), device_id_type=pl.DeviceIdType.MESH,
            )

            @pl.when(d != my_pos)
            def _():
                ag_rdma.start()

            ag_rdmas.append((d, ag_rdma))

        for s in range(N_DEV):
            ag_rx = pltpu.make_async_remote_copy(
                src_ref=out_ref.at[:, pl.ds(s * C, C), :],
                dst_ref=out_ref.at[:, pl.ds(s * C, C), :],
                send_sem=ag_send.at[s], recv_sem=ag_recv.at[s],
                device_id=(s,), device_id_type=pl.DeviceIdType.MESH,
            )

            @pl.when(s != my_pos)
            def _():
                ag_rx.wait_recv()

        for d, o_rdma, l_rdma in rs_rdmas:
            @pl.when(d != my_pos)
            def _():
                o_rdma.wait_send()
                l_rdma.wait_send()
        for d, ag_rdma in ag_rdmas:
            @pl.when(d != my_pos)
            def _():
                ag_rdma.wait_send()

    return pl.pallas_call(
        body,
        out_shape=jax.ShapeDtypeStruct((B, Sq, D), jnp.float32),
        in_specs=[pl.BlockSpec(memory_space=pltpu.VMEM)] * 5,
        out_specs=pl.BlockSpec(memory_space=pltpu.VMEM),
        scratch_shapes=[
            pltpu.VMEM((B, Sq, Dq), jnp.float32),
            pltpu.VMEM((B, Sq, Dq), jnp.float32),
            pltpu.VMEM((Sq, B * Hq), jnp.float32),
            pltpu.VMEM((B, C, Dq), jnp.float32),
            pltpu.VMEM((N_DEV, B, C, Dq), jnp.float32),
            pltpu.VMEM((N_DEV, C, B * Hq), jnp.float32),
            pltpu.SemaphoreType.DMA((N_DEV,)),
            pltpu.SemaphoreType.DMA((N_DEV,)),
            pltpu.SemaphoreType.DMA((N_DEV,)),
            pltpu.SemaphoreType.DMA((N_DEV,)),
            pltpu.SemaphoreType.DMA((N_DEV,)),
            pltpu.SemaphoreType.DMA((N_DEV,)),
        ],
        compiler_params=pltpu.CompilerParams(
            collective_id=0, vmem_limit_bytes=110 * 1024 * 1024),
    )(x, Wq, Wo, K_ext, V_ext)
